# baseline (speedup 1.0000x reference)
import jax
import jax.numpy as jnp
from jax.experimental import pallas as pl
from jax.experimental.pallas import tpu as pltpu

_N = 10000
_D = 64
_K = 20
_R = 80
_W = 10240


def _split2(a):
    a1 = a.astype(jnp.bfloat16)
    a2 = (a - a1.astype(jnp.float32)).astype(jnp.bfloat16)
    return a1, a2


def _split3(a):
    a1 = a.astype(jnp.bfloat16)
    r = a - a1.astype(jnp.float32)
    a2 = r.astype(jnp.bfloat16)
    a3 = (r - a2.astype(jnp.float32)).astype(jnp.bfloat16)
    return a1, a2, a3


def _mm(a, b):
    return jnp.dot(a, b, preferred_element_type=jnp.float32)


def _dot_x6(a, b):
    a1, a2, a3 = _split3(a)
    b1, b2, b3 = _split3(b)
    lo = _mm(a3, b1) + _mm(a2, b2) + _mm(a1, b3)
    mid = _mm(a2, b1) + _mm(a1, b2)
    return (lo + mid) + _mm(a1, b1)


def _dot_x3(a, b):
    a1, a2 = _split2(a)
    b1, b2 = _split2(b)
    return (_mm(a2, b1) + _mm(a1, b2)) + _mm(a1, b1)


def _pre_kernel(x_ref, wl_ref, bl_ref, emb_ref, ati_ref, atj_ref, atei_ref,
                atej_ref, h_ref, inv_ref, ai_ref, aj_ref):
    h = _mm(x_ref[:].astype(jnp.bfloat16),
            wl_ref[:].astype(jnp.bfloat16)) + bl_ref[:]
    h_ref[:] = h
    e = emb_ref[:]
    nrm2 = jnp.maximum(jnp.sum(e * e, axis=1, keepdims=True), 1e-24)
    r = jax.lax.rsqrt(nrm2)
    r = r * (1.5 - 0.5 * nrm2 * r * r)
    inv_ref[:] = jnp.minimum(r, 1e12)
    ai_ref[:] = (jnp.sum(h * ati_ref[:], axis=1, keepdims=True)
                 + jnp.sum(e * atei_ref[:], axis=1, keepdims=True))
    aj_ref[:] = (jnp.sum(h * atj_ref[:], axis=1, keepdims=True)
                 + jnp.sum(e * atej_ref[:], axis=1, keepdims=True))


def _attn_kernel(emb_t_ref, embT_ref, ai_t_ref, ajr_ref, h_ref, gb_ref,
                 invr_ref, o_ref):
    iota = jax.lax.broadcasted_iota(jnp.int32, (_R, _W), 1)
    s = _mm(emb_t_ref[:], embT_ref[:]) * invr_ref[:]
    s = jnp.where(iota >= _N, -jnp.inf, s)
    bm = jnp.max(s.reshape(_R, _W // 32, 32), axis=2)
    t = None
    for _ in range(_K):
        t = jnp.max(bm, axis=1, keepdims=True)
        bm = jnp.where(bm >= t, -jnp.inf, bm)
    cm = s >= jnp.maximum(t, -3.0e38)
    s_c = jnp.where(cm, s, jnp.inf)
    cnt = jnp.sum(cm.astype(jnp.int32), axis=1, keepdims=True)

    def _drop_min(carry):
        s_cc, cnt_c = carry
        need = cnt_c > _K
        m_c = jnp.min(s_cc, axis=1, keepdims=True)
        key = jnp.where(s_cc == m_c, iota, -1)
        p = jnp.max(key, axis=1, keepdims=True)
        p = jnp.where(need, p, -1)
        s_cc = jnp.where(iota == p, jnp.inf, s_cc)
        return s_cc, cnt_c - need.astype(jnp.int32)

    for _ in range(8):
        s_c, cnt = _drop_min((s_c, cnt))
    s_c, cnt = jax.lax.while_loop(
        lambda c: jnp.any(c[1] > _K), _drop_min, (s_c, cnt))
    sel = s_c != jnp.inf
    alpha = ai_t_ref[:] + ajr_ref[:]
    alpha = jnp.where(alpha >= 0, alpha, 0.2 * alpha)
    af = jnp.where(sel, alpha, -jnp.inf)
    m2 = jnp.max(af, axis=1, keepdims=True)
    ex = jnp.exp(af - m2)
    ssum = jnp.sum(ex, axis=1, keepdims=True)
    rs = 1.0 / (ssum + 1e-16)
    o_ref[:] = _dot_x3(ex, h_ref[:]) * rs + gb_ref[:]


def _post_kernel(o_ref, emb_ref, g1_ref, b1_ref, g2_ref, b2_ref, wo_ref,
                 bo_ref, z_ref):
    o = o_ref[:]
    mu = jnp.mean(o, axis=0, keepdims=True)
    var = jnp.mean((o - mu) * (o - mu), axis=0, keepdims=True)
    o = (o - mu) * jax.lax.rsqrt(var + 1e-5) * g1_ref[:] + b1_ref[:]
    o = jnp.maximum(o, 0.0)
    y = o * emb_ref[:]
    mu2 = jnp.mean(y, axis=0, keepdims=True)
    var2 = jnp.mean((y - mu2) * (y - mu2), axis=0, keepdims=True)
    y = (y - mu2) * jax.lax.rsqrt(var2 + 1e-5) * g2_ref[:] + b2_ref[:]
    y = jnp.maximum(y, 0.0)
    z = _mm(y.astype(jnp.bfloat16),
            wo_ref[:].astype(jnp.bfloat16)) + bo_ref[:]
    z_ref[:] = jnp.tanh(z)


def kernel(x, edge_index, emb, W_lin, b_lin, att_i, att_j, att_em_i, att_em_j,
           gnn_bias, bn1_gamma, bn1_beta, bn2_gamma, bn2_beta, W_out, b_out):
    del edge_index
    b, n, f = x.shape
    p_out = W_out.shape[1]
    xf = x.reshape(n, f)

    rp = 1000 if n % 1000 == 0 else n
    h, inv, ai, aj = pl.pallas_call(
        _pre_kernel,
        grid=(n // rp,),
        in_specs=[
            pl.BlockSpec((rp, f), lambda i: (i, 0)),
            pl.BlockSpec((f, _D), lambda i: (0, 0)),
            pl.BlockSpec((1, _D), lambda i: (0, 0)),
            pl.BlockSpec((rp, _D), lambda i: (i, 0)),
            pl.BlockSpec((1, _D), lambda i: (0, 0)),
            pl.BlockSpec((1, _D), lambda i: (0, 0)),
            pl.BlockSpec((1, _D), lambda i: (0, 0)),
            pl.BlockSpec((1, _D), lambda i: (0, 0)),
        ],
        out_specs=[
            pl.BlockSpec((rp, _D), lambda i: (i, 0)),
            pl.BlockSpec((rp, 1), lambda i: (i, 0)),
            pl.BlockSpec((rp, 1), lambda i: (i, 0)),
            pl.BlockSpec((rp, 1), lambda i: (i, 0)),
        ],
        out_shape=[
            jax.ShapeDtypeStruct((n, _D), jnp.float32),
            jax.ShapeDtypeStruct((n, 1), jnp.float32),
            jax.ShapeDtypeStruct((n, 1), jnp.float32),
            jax.ShapeDtypeStruct((n, 1), jnp.float32),
        ],
        compiler_params=pltpu.CompilerParams(
            dimension_semantics=("parallel",)),
    )(xf, W_lin, b_lin.reshape(1, _D), emb, att_i.reshape(1, _D),
      att_j.reshape(1, _D), att_em_i.reshape(1, _D), att_em_j.reshape(1, _D))

    emb_bf = emb.astype(jnp.bfloat16)
    pad = _W - n
    embT_bf = jnp.pad(emb_bf.T, ((0, 0), (0, pad)))
    ajr = jnp.pad(aj.reshape(1, n), ((0, 0), (0, pad)))
    invr = jnp.pad(inv.reshape(1, n), ((0, 0), (0, pad)))
    h_pad = jnp.pad(h, ((0, pad), (0, 0)))

    out = pl.pallas_call(
        _attn_kernel,
        grid=(n // _R,),
        in_specs=[
            pl.BlockSpec((_R, _D), lambda i: (i, 0)),
            pl.BlockSpec((_D, _W), lambda i: (0, 0)),
            pl.BlockSpec((_R, 1), lambda i: (i, 0)),
            pl.BlockSpec((1, _W), lambda i: (0, 0)),
            pl.BlockSpec((_W, _D), lambda i: (0, 0)),
            pl.BlockSpec((1, _D), lambda i: (0, 0)),
            pl.BlockSpec((1, _W), lambda i: (0, 0)),
        ],
        out_specs=pl.BlockSpec((_R, _D), lambda i: (i, 0)),
        out_shape=jax.ShapeDtypeStruct((n, _D), jnp.float32),
        compiler_params=pltpu.CompilerParams(
            dimension_semantics=("parallel",)),
    )(emb_bf, embT_bf, ai, ajr, h_pad, gnn_bias.reshape(1, _D), invr)

    z = pl.pallas_call(
        _post_kernel,
        out_shape=jax.ShapeDtypeStruct((n, p_out), jnp.float32),
    )(out, emb, bn1_gamma.reshape(1, _D), bn1_beta.reshape(1, _D),
      bn2_gamma.reshape(1, _D), bn2_beta.reshape(1, _D), W_out,
      b_out.reshape(1, p_out))

    return z.reshape(b, n, p_out)


# speedup vs baseline: 2.7227x; 2.7227x over previous
import jax
import jax.numpy as jnp
from jax.experimental import pallas as pl
from jax.experimental.pallas import tpu as pltpu

_N = 10000
_D = 64
_K = 20
_R = 200
_W = 10240


def _split2(a):
    a1 = a.astype(jnp.bfloat16)
    a2 = (a - a1.astype(jnp.float32)).astype(jnp.bfloat16)
    return a1, a2


def _split3(a):
    a1 = a.astype(jnp.bfloat16)
    r = a - a1.astype(jnp.float32)
    a2 = r.astype(jnp.bfloat16)
    a3 = (r - a2.astype(jnp.float32)).astype(jnp.bfloat16)
    return a1, a2, a3


def _mm(a, b):
    return jnp.dot(a, b, preferred_element_type=jnp.float32)


def _dot_x6(a, b):
    a1, a2, a3 = _split3(a)
    b1, b2, b3 = _split3(b)
    lo = _mm(a3, b1) + _mm(a2, b2) + _mm(a1, b3)
    mid = _mm(a2, b1) + _mm(a1, b2)
    return (lo + mid) + _mm(a1, b1)


def _dot_x3(a, b):
    a1, a2 = _split2(a)
    b1, b2 = _split2(b)
    return (_mm(a2, b1) + _mm(a1, b2)) + _mm(a1, b1)


def _pre_kernel(x_ref, wl_ref, bl_ref, emb_ref, ati_ref, atj_ref, atei_ref,
                atej_ref, h_ref, inv_ref, ai_ref, aj_ref):
    h = _mm(x_ref[:].astype(jnp.bfloat16),
            wl_ref[:].astype(jnp.bfloat16)) + bl_ref[:]
    h_ref[:] = h
    e = emb_ref[:]
    nrm2 = jnp.maximum(jnp.sum(e * e, axis=1, keepdims=True), 1e-24)
    r = jax.lax.rsqrt(nrm2)
    r = r * (1.5 - 0.5 * nrm2 * r * r)
    inv_ref[:] = jnp.minimum(r, 1e12)
    ai_ref[:] = (jnp.sum(h * ati_ref[:], axis=1, keepdims=True)
                 + jnp.sum(e * atei_ref[:], axis=1, keepdims=True))
    aj_ref[:] = (jnp.sum(h * atj_ref[:], axis=1, keepdims=True)
                 + jnp.sum(e * atej_ref[:], axis=1, keepdims=True))


def _attn_kernel(emb_t_ref, embT_ref, ai_t_ref, ajr_ref, h_ref, gb_ref,
                 invr_ref, o_ref):
    pm = jax.lax.broadcasted_iota(jnp.int32, (1, _W), 1) >= _N
    s = _mm(emb_t_ref[:], embT_ref[:]) * invr_ref[:]
    s = jnp.where(pm, -jnp.inf, s)
    bm = jnp.max(s.reshape(_R, _W // 128, 128), axis=2)
    tb = None
    for _ in range(_K):
        tb = jnp.max(bm, axis=1, keepdims=True)
        bm = jnp.where(bm >= tb, -jnp.inf, bm)
    lo = jnp.maximum(tb, -3.0e38)
    m1 = jnp.max(s, axis=1, keepdims=True)
    hi = m1 + jnp.maximum(jnp.abs(m1) * 1e-3, 1e-6)
    t_found = lo
    done = jnp.zeros((_R, 1), jnp.bool_)
    for _ in range(24):
        mid = 0.5 * (lo + hi)
        c = jnp.sum((s >= mid).astype(jnp.float32), axis=1, keepdims=True)
        live = jnp.logical_not(done)
        hit = jnp.logical_and(c == float(_K), live)
        t_found = jnp.where(hit, mid, t_found)
        done = jnp.logical_or(done, hit)
        live = jnp.logical_not(done)
        ge = c >= float(_K)
        lo = jnp.where(jnp.logical_and(ge, live), mid, lo)
        hi = jnp.where(jnp.logical_and(jnp.logical_not(ge), live), mid, hi)
    tt = jnp.where(done, t_found, lo)
    cm = s >= tt
    s_c = jnp.where(cm, s, jnp.inf)
    cnt = jnp.sum(cm.astype(jnp.int32), axis=1, keepdims=True)

    def _drop_min(carry):
        s_cc, cnt_c = carry
        iota = jax.lax.broadcasted_iota(jnp.int32, (_R, _W), 1)
        need = cnt_c > _K
        m_c = jnp.min(s_cc, axis=1, keepdims=True)
        key = jnp.where(s_cc == m_c, iota, -1)
        p = jnp.max(key, axis=1, keepdims=True)
        p = jnp.where(need, p, -1)
        s_cc = jnp.where(iota == p, jnp.inf, s_cc)
        return s_cc, cnt_c - need.astype(jnp.int32)

    s_c, cnt = jax.lax.while_loop(
        lambda c_: jnp.any(c_[1] > _K), _drop_min, (s_c, cnt))
    sel = s_c != jnp.inf
    alpha = ai_t_ref[:] + ajr_ref[:]
    alpha = jnp.where(alpha >= 0, alpha, 0.2 * alpha)
    af = jnp.where(sel, alpha, -jnp.inf)
    m2 = jnp.max(af, axis=1, keepdims=True)
    ex = jnp.exp(af - m2)
    ssum = jnp.sum(ex, axis=1, keepdims=True)
    rs = 1.0 / (ssum + 1e-16)
    o_ref[:] = _dot_x3(ex, h_ref[:]) * rs + gb_ref[:]


def _post_kernel(o_ref, emb_ref, g1_ref, b1_ref, g2_ref, b2_ref, wo_ref,
                 bo_ref, z_ref):
    o = o_ref[:]
    mu = jnp.mean(o, axis=0, keepdims=True)
    var = jnp.mean((o - mu) * (o - mu), axis=0, keepdims=True)
    o = (o - mu) * jax.lax.rsqrt(var + 1e-5) * g1_ref[:] + b1_ref[:]
    o = jnp.maximum(o, 0.0)
    y = o * emb_ref[:]
    mu2 = jnp.mean(y, axis=0, keepdims=True)
    var2 = jnp.mean((y - mu2) * (y - mu2), axis=0, keepdims=True)
    y = (y - mu2) * jax.lax.rsqrt(var2 + 1e-5) * g2_ref[:] + b2_ref[:]
    y = jnp.maximum(y, 0.0)
    z = _mm(y.astype(jnp.bfloat16),
            wo_ref[:].astype(jnp.bfloat16)) + bo_ref[:]
    z_ref[:] = jnp.tanh(z)


def kernel(x, edge_index, emb, W_lin, b_lin, att_i, att_j, att_em_i, att_em_j,
           gnn_bias, bn1_gamma, bn1_beta, bn2_gamma, bn2_beta, W_out, b_out):
    del edge_index
    b, n, f = x.shape
    p_out = W_out.shape[1]
    xf = x.reshape(n, f)

    rp = 1000 if n % 1000 == 0 else n
    h, inv, ai, aj = pl.pallas_call(
        _pre_kernel,
        grid=(n // rp,),
        in_specs=[
            pl.BlockSpec((rp, f), lambda i: (i, 0)),
            pl.BlockSpec((f, _D), lambda i: (0, 0)),
            pl.BlockSpec((1, _D), lambda i: (0, 0)),
            pl.BlockSpec((rp, _D), lambda i: (i, 0)),
            pl.BlockSpec((1, _D), lambda i: (0, 0)),
            pl.BlockSpec((1, _D), lambda i: (0, 0)),
            pl.BlockSpec((1, _D), lambda i: (0, 0)),
            pl.BlockSpec((1, _D), lambda i: (0, 0)),
        ],
        out_specs=[
            pl.BlockSpec((rp, _D), lambda i: (i, 0)),
            pl.BlockSpec((rp, 1), lambda i: (i, 0)),
            pl.BlockSpec((rp, 1), lambda i: (i, 0)),
            pl.BlockSpec((rp, 1), lambda i: (i, 0)),
        ],
        out_shape=[
            jax.ShapeDtypeStruct((n, _D), jnp.float32),
            jax.ShapeDtypeStruct((n, 1), jnp.float32),
            jax.ShapeDtypeStruct((n, 1), jnp.float32),
            jax.ShapeDtypeStruct((n, 1), jnp.float32),
        ],
        compiler_params=pltpu.CompilerParams(
            dimension_semantics=("parallel",)),
    )(xf, W_lin, b_lin.reshape(1, _D), emb, att_i.reshape(1, _D),
      att_j.reshape(1, _D), att_em_i.reshape(1, _D), att_em_j.reshape(1, _D))

    emb_bf = emb.astype(jnp.bfloat16)
    pad = _W - n
    embT_bf = jnp.pad(emb_bf.T, ((0, 0), (0, pad)))
    ajr = jnp.pad(aj.reshape(1, n), ((0, 0), (0, pad)))
    invr = jnp.pad(inv.reshape(1, n), ((0, 0), (0, pad)))
    h_pad = jnp.pad(h, ((0, pad), (0, 0)))

    out = pl.pallas_call(
        _attn_kernel,
        grid=(n // _R,),
        in_specs=[
            pl.BlockSpec((_R, _D), lambda i: (i, 0)),
            pl.BlockSpec((_D, _W), lambda i: (0, 0)),
            pl.BlockSpec((_R, 1), lambda i: (i, 0)),
            pl.BlockSpec((1, _W), lambda i: (0, 0)),
            pl.BlockSpec((_W, _D), lambda i: (0, 0)),
            pl.BlockSpec((1, _D), lambda i: (0, 0)),
            pl.BlockSpec((1, _W), lambda i: (0, 0)),
        ],
        out_specs=pl.BlockSpec((_R, _D), lambda i: (i, 0)),
        out_shape=jax.ShapeDtypeStruct((n, _D), jnp.float32),
        compiler_params=pltpu.CompilerParams(
            dimension_semantics=("parallel",)),
    )(emb_bf, embT_bf, ai, ajr, h_pad, gnn_bias.reshape(1, _D), invr)

    z = pl.pallas_call(
        _post_kernel,
        out_shape=jax.ShapeDtypeStruct((n, p_out), jnp.float32),
    )(out, emb, bn1_gamma.reshape(1, _D), bn1_beta.reshape(1, _D),
      bn2_gamma.reshape(1, _D), bn2_beta.reshape(1, _D), W_out,
      b_out.reshape(1, p_out))

    return z.reshape(b, n, p_out)


# speedup vs baseline: 13.3948x; 4.9197x over previous
import jax
import jax.numpy as jnp
from jax.experimental import pallas as pl
from jax.experimental.pallas import tpu as pltpu

_N = 10000
_D = 64
_K = 20
_R = 200
_W = 10240


def _split2(a):
    a1 = a.astype(jnp.bfloat16)
    a2 = (a - a1.astype(jnp.float32)).astype(jnp.bfloat16)
    return a1, a2


def _split3(a):
    a1 = a.astype(jnp.bfloat16)
    r = a - a1.astype(jnp.float32)
    a2 = r.astype(jnp.bfloat16)
    a3 = (r - a2.astype(jnp.float32)).astype(jnp.bfloat16)
    return a1, a2, a3


def _mm(a, b):
    return jnp.dot(a, b, preferred_element_type=jnp.float32)


def _dot_x6(a, b):
    a1, a2, a3 = _split3(a)
    b1, b2, b3 = _split3(b)
    lo = _mm(a3, b1) + _mm(a2, b2) + _mm(a1, b3)
    mid = _mm(a2, b1) + _mm(a1, b2)
    return (lo + mid) + _mm(a1, b1)


def _dot_x3(a, b):
    a1, a2 = _split2(a)
    b1, b2 = _split2(b)
    return (_mm(a2, b1) + _mm(a1, b2)) + _mm(a1, b1)


def _pre_kernel(x_ref, wl_ref, bl_ref, emb_ref, ati_ref, atj_ref, atei_ref,
                atej_ref, h_ref, inv_ref, ai_ref, aj_ref):
    h = _mm(x_ref[:].astype(jnp.bfloat16),
            wl_ref[:].astype(jnp.bfloat16)) + bl_ref[:]
    h_ref[:] = h
    e = emb_ref[:]
    nrm2 = jnp.maximum(jnp.sum(e * e, axis=1, keepdims=True), 1e-24)
    r = jax.lax.rsqrt(nrm2)
    r = r * (1.5 - 0.5 * nrm2 * r * r)
    inv_ref[:] = jnp.minimum(r, 1e12)
    ai_ref[:] = (jnp.sum(h * ati_ref[:], axis=1, keepdims=True)
                 + jnp.sum(e * atei_ref[:], axis=1, keepdims=True))
    aj_ref[:] = (jnp.sum(h * atj_ref[:], axis=1, keepdims=True)
                 + jnp.sum(e * atej_ref[:], axis=1, keepdims=True))


def _attn_kernel(emb_t_ref, embT_ref, ai_t_ref, ajr_ref, h_ref, gb_ref,
                 invr_ref, o_ref):
    pm = jax.lax.broadcasted_iota(jnp.int32, (1, _W), 1) >= _N
    s = _mm(emb_t_ref[:], embT_ref[:]) * invr_ref[:]
    s = jnp.where(pm, -jnp.inf, s)
    nblk = _W // 128
    work = [s[:, 128 * k:128 * (k + 1)] for k in range(nblk)]
    cands = []
    for level in range(6):
        t_arr = work
        while len(t_arr) > 1:
            nxt = [jnp.maximum(t_arr[i], t_arr[i + 1])
                   for i in range(0, len(t_arr) - 1, 2)]
            if len(t_arr) % 2:
                nxt.append(t_arr[-1])
            t_arr = nxt
        m = t_arr[0]
        cands.append(m)
        if level < 5:
            work = [jnp.where(w >= m, -jnp.inf, w) for w in work]
    cand = jnp.concatenate(cands, axis=1)
    t = None
    for _ in range(_K):
        t = jnp.max(cand, axis=1, keepdims=True)
        cand = jnp.where(cand >= t, -jnp.inf, cand)
    iota_r = jax.lax.broadcasted_iota(jnp.int32, (1, _W), 1)

    def _count(pred):
        return jnp.sum(pred.astype(jnp.int32), axis=1, keepdims=True)

    cgt = _count(s > t)
    cge = _count(s >= t)

    def _body_raise(c):
        t_, cgt_, _ = c
        need = cgt_ >= _K
        up = jnp.min(jnp.where(s > t_, s, jnp.inf), axis=1, keepdims=True)
        t2 = jnp.where(need, up, t_)
        return t2, _count(s > t2), _count(s >= t2)

    t, cgt, cge = jax.lax.while_loop(
        lambda c: jnp.any(c[1] >= _K), _body_raise, (t, cgt, cge))

    def _body_lower(c):
        t_, _, cge_ = c
        need = cge_ < _K
        dn = jnp.max(jnp.where(s >= t_, -jnp.inf, s), axis=1, keepdims=True)
        t2 = jnp.where(need, dn, t_)
        return t2, _count(s > t2), _count(s >= t2)

    t, cgt, cge = jax.lax.while_loop(
        lambda c: jnp.any(c[2] < _K), _body_lower, (t, cgt, cge))

    cut0 = jnp.full((_R, 1), _W, jnp.int32)

    def _body_trim(c):
        cut_, cnt_ = c
        need = cnt_ > _K
        inc = jnp.logical_and(s == t, iota_r <= cut_)
        p = jnp.max(jnp.where(inc, iota_r, -1), axis=1, keepdims=True)
        cut2 = jnp.where(need, p - 1, cut_)
        return cut2, cnt_ - need.astype(jnp.int32)

    cut, _ = jax.lax.while_loop(
        lambda c: jnp.any(c[1] > _K), _body_trim, (cut0, cge))
    sel = jnp.logical_and(
        s >= t, jnp.logical_or(s != t, iota_r <= cut))
    alpha = ai_t_ref[:] + ajr_ref[:]
    alpha = jnp.where(alpha >= 0, alpha, 0.2 * alpha)
    af = jnp.where(sel, alpha, -jnp.inf)
    m2 = jnp.max(af, axis=1, keepdims=True)
    ex = jnp.exp(af - m2)
    ssum = jnp.sum(ex, axis=1, keepdims=True)
    rs = 1.0 / (ssum + 1e-16)
    o_ref[:] = _dot_x3(ex, h_ref[:]) * rs + gb_ref[:]


def _post_kernel(o_ref, emb_ref, g1_ref, b1_ref, g2_ref, b2_ref, wo_ref,
                 bo_ref, z_ref):
    o = o_ref[:]
    mu = jnp.mean(o, axis=0, keepdims=True)
    var = jnp.mean((o - mu) * (o - mu), axis=0, keepdims=True)
    o = (o - mu) * jax.lax.rsqrt(var + 1e-5) * g1_ref[:] + b1_ref[:]
    o = jnp.maximum(o, 0.0)
    y = o * emb_ref[:]
    mu2 = jnp.mean(y, axis=0, keepdims=True)
    var2 = jnp.mean((y - mu2) * (y - mu2), axis=0, keepdims=True)
    y = (y - mu2) * jax.lax.rsqrt(var2 + 1e-5) * g2_ref[:] + b2_ref[:]
    y = jnp.maximum(y, 0.0)
    z = _mm(y.astype(jnp.bfloat16),
            wo_ref[:].astype(jnp.bfloat16)) + bo_ref[:]
    z_ref[:] = jnp.tanh(z)


def kernel(x, edge_index, emb, W_lin, b_lin, att_i, att_j, att_em_i, att_em_j,
           gnn_bias, bn1_gamma, bn1_beta, bn2_gamma, bn2_beta, W_out, b_out):
    del edge_index
    b, n, f = x.shape
    p_out = W_out.shape[1]
    xf = x.reshape(n, f)

    rp = 1000 if n % 1000 == 0 else n
    h, inv, ai, aj = pl.pallas_call(
        _pre_kernel,
        grid=(n // rp,),
        in_specs=[
            pl.BlockSpec((rp, f), lambda i: (i, 0)),
            pl.BlockSpec((f, _D), lambda i: (0, 0)),
            pl.BlockSpec((1, _D), lambda i: (0, 0)),
            pl.BlockSpec((rp, _D), lambda i: (i, 0)),
            pl.BlockSpec((1, _D), lambda i: (0, 0)),
            pl.BlockSpec((1, _D), lambda i: (0, 0)),
            pl.BlockSpec((1, _D), lambda i: (0, 0)),
            pl.BlockSpec((1, _D), lambda i: (0, 0)),
        ],
        out_specs=[
            pl.BlockSpec((rp, _D), lambda i: (i, 0)),
            pl.BlockSpec((rp, 1), lambda i: (i, 0)),
            pl.BlockSpec((rp, 1), lambda i: (i, 0)),
            pl.BlockSpec((rp, 1), lambda i: (i, 0)),
        ],
        out_shape=[
            jax.ShapeDtypeStruct((n, _D), jnp.float32),
            jax.ShapeDtypeStruct((n, 1), jnp.float32),
            jax.ShapeDtypeStruct((n, 1), jnp.float32),
            jax.ShapeDtypeStruct((n, 1), jnp.float32),
        ],
        compiler_params=pltpu.CompilerParams(
            dimension_semantics=("parallel",)),
    )(xf, W_lin, b_lin.reshape(1, _D), emb, att_i.reshape(1, _D),
      att_j.reshape(1, _D), att_em_i.reshape(1, _D), att_em_j.reshape(1, _D))

    emb_bf = emb.astype(jnp.bfloat16)
    pad = _W - n
    embT_bf = jnp.pad(emb_bf.T, ((0, 0), (0, pad)))
    ajr = jnp.pad(aj.reshape(1, n), ((0, 0), (0, pad)))
    invr = jnp.pad(inv.reshape(1, n), ((0, 0), (0, pad)))
    h_pad = jnp.pad(h, ((0, pad), (0, 0)))

    out = pl.pallas_call(
        _attn_kernel,
        grid=(n // _R,),
        in_specs=[
            pl.BlockSpec((_R, _D), lambda i: (i, 0)),
            pl.BlockSpec((_D, _W), lambda i: (0, 0)),
            pl.BlockSpec((_R, 1), lambda i: (i, 0)),
            pl.BlockSpec((1, _W), lambda i: (0, 0)),
            pl.BlockSpec((_W, _D), lambda i: (0, 0)),
            pl.BlockSpec((1, _D), lambda i: (0, 0)),
            pl.BlockSpec((1, _W), lambda i: (0, 0)),
        ],
        out_specs=pl.BlockSpec((_R, _D), lambda i: (i, 0)),
        out_shape=jax.ShapeDtypeStruct((n, _D), jnp.float32),
        compiler_params=pltpu.CompilerParams(
            dimension_semantics=("parallel",)),
    )(emb_bf, embT_bf, ai, ajr, h_pad, gnn_bias.reshape(1, _D), invr)

    z = pl.pallas_call(
        _post_kernel,
        out_shape=jax.ShapeDtypeStruct((n, p_out), jnp.float32),
    )(out, emb, bn1_gamma.reshape(1, _D), bn1_beta.reshape(1, _D),
      bn2_gamma.reshape(1, _D), bn2_beta.reshape(1, _D), W_out,
      b_out.reshape(1, p_out))

    return z.reshape(b, n, p_out)
